# baseline (speedup 1.0000x reference)
import jax
import jax.numpy as jnp
from jax import lax
from jax.experimental import pallas as pl

_STRIPS = 4
_TB = 256


def _strip_argmin(d, k):
    w = k // _STRIPS
    acc = jnp.full((d.shape[0], 1), jnp.inf, jnp.float32)
    idx = jnp.zeros((d.shape[0], 1), jnp.int32)
    for c in range(_STRIPS):
        strip = d[:, c * w:(c + 1) * w]
        m = jnp.min(strip, axis=1, keepdims=True)
        iota = lax.broadcasted_iota(jnp.int32, strip.shape, 1) + c * w
        ic = jnp.min(jnp.where(strip == m, iota, k), axis=1, keepdims=True)
        win = m < acc
        idx = jnp.where(win, ic, idx)
        acc = jnp.where(win, m.astype(jnp.bfloat16).astype(jnp.float32), acc)
    return idx


def _distances(x, e, xn, en_row):
    s = lax.dot_general(x.astype(jnp.bfloat16), e.astype(jnp.bfloat16),
                        (((1,), (1,)), ((), ())),
                        preferred_element_type=jnp.float32)
    return (xn - 2.0 * s) + en_row


def _gather_rows(idx, e):
    iota = lax.broadcasted_iota(jnp.int32, (idx.shape[0], e.shape[0]), 1)
    onehot = (iota == idx).astype(jnp.float32)
    return lax.dot_general(onehot, e, (((1,), (0,)), ((), ())),
                           preferred_element_type=jnp.float32,
                           precision=lax.Precision.HIGHEST)


def _stage1_body(x_ref, e_ref, xn_ref, en_ref, i_ref, q_ref, r_ref):
    x = x_ref[...]
    e = e_ref[...]
    d = _distances(x, e, xn_ref[...], en_ref[...])
    idx = _strip_argmin(d, e.shape[0])
    q = _gather_rows(idx, e)
    i_ref[...] = idx
    q_ref[...] = q
    r_ref[...] = x - q


def _stage2_body(r_ref, e_ref, xn_ref, en_ref, q1_ref, i_ref, out_ref):
    r = r_ref[...]
    e = e_ref[...]
    d = _distances(r, e, xn_ref[...], en_ref[...])
    idx = _strip_argmin(d, e.shape[0])
    q2 = _gather_rows(idx, e)
    i_ref[...] = idx
    out_ref[...] = q1_ref[...] + q2


def kernel(z, embed):
    b, t, dm = z.shape
    k = embed.shape[0]
    n = b * t
    flat = z.reshape(n, dm)
    en_row = jnp.sum(embed * embed, axis=1)[None, :]
    xn1 = jnp.sum(flat * flat, axis=1, keepdims=True)

    grid = (n // _TB,)
    tok_spec = pl.BlockSpec((_TB, dm), lambda i: (i, 0))
    e_spec = pl.BlockSpec((k, dm), lambda i: (0, 0))
    xn_spec = pl.BlockSpec((_TB, 1), lambda i: (i, 0))
    en_spec = pl.BlockSpec((1, k), lambda i: (0, 0))
    idx_spec = pl.BlockSpec((_TB, 1), lambda i: (i, 0))

    i1, q1, r1 = pl.pallas_call(
        _stage1_body,
        grid=grid,
        in_specs=[tok_spec, e_spec, xn_spec, en_spec],
        out_specs=[idx_spec, tok_spec, tok_spec],
        out_shape=[
            jax.ShapeDtypeStruct((n, 1), jnp.int32),
            jax.ShapeDtypeStruct((n, dm), jnp.float32),
            jax.ShapeDtypeStruct((n, dm), jnp.float32),
        ],
    )(flat, embed, xn1, en_row)

    xn2 = jnp.sum(r1 * r1, axis=1, keepdims=True)

    i2, out = pl.pallas_call(
        _stage2_body,
        grid=grid,
        in_specs=[tok_spec, e_spec, xn_spec, en_spec, tok_spec],
        out_specs=[idx_spec, tok_spec],
        out_shape=[
            jax.ShapeDtypeStruct((n, 1), jnp.int32),
            jax.ShapeDtypeStruct((n, dm), jnp.float32),
        ],
    )(r1, embed, xn2, en_row, q1)

    return (out.reshape(b, t, dm), i1.reshape(b, t).astype(jnp.int32),
            i2.reshape(b, t))


# speedup vs baseline: 1.0608x; 1.0608x over previous
import jax
import jax.numpy as jnp
from jax import lax
from jax.experimental import pallas as pl

_STRIPS = 4
_TB = 256


_LANES = 128


def _strip_argmin(d, k):
    tb = d.shape[0]
    w = k // _STRIPS
    lane = lax.broadcasted_iota(jnp.int32, (tb, _LANES), 1)
    acc = jnp.full((tb, 1), jnp.inf, jnp.float32)
    idx = jnp.zeros((tb, 1), jnp.int32)
    for c in range(_STRIPS):
        vacc = d[:, c * w:c * w + _LANES]
        cacc = jnp.zeros((tb, _LANES), jnp.int32)
        for j in range(1, w // _LANES):
            v = d[:, c * w + j * _LANES:c * w + (j + 1) * _LANES]
            win = v < vacc
            vacc = jnp.minimum(vacc, v)
            cacc = jnp.where(win, j, cacc)
        gidx = c * w + cacc * _LANES + lane
        m = jnp.min(vacc, axis=1, keepdims=True)
        ic = jnp.min(jnp.where(vacc == m, gidx, k), axis=1, keepdims=True)
        win = m < acc
        idx = jnp.where(win, ic, idx)
        acc = jnp.where(win, m.astype(jnp.bfloat16).astype(jnp.float32), acc)
    return idx


def _distances(x, e, xn, en_row):
    s = lax.dot_general((-2.0 * x).astype(jnp.bfloat16), e.astype(jnp.bfloat16),
                        (((1,), (1,)), ((), ())),
                        preferred_element_type=jnp.float32)
    return (xn + s) + en_row


def _gather_rows(idx, e):
    iota = lax.broadcasted_iota(jnp.int32, (idx.shape[0], e.shape[0]), 1)
    onehot = (iota == idx).astype(jnp.float32)
    return lax.dot_general(onehot, e, (((1,), (0,)), ((), ())),
                           preferred_element_type=jnp.float32,
                           precision=lax.Precision.HIGHEST)


def _stage1_body(x_ref, e_ref, xn_ref, en_ref, i_ref, q_ref, r_ref):
    x = x_ref[...]
    e = e_ref[...]
    d = _distances(x, e, xn_ref[...], en_ref[...])
    idx = _strip_argmin(d, e.shape[0])
    q = _gather_rows(idx, e)
    i_ref[...] = idx
    q_ref[...] = q
    r_ref[...] = x - q


def _stage2_body(r_ref, e_ref, xn_ref, en_ref, q1_ref, i_ref, out_ref):
    r = r_ref[...]
    e = e_ref[...]
    d = _distances(r, e, xn_ref[...], en_ref[...])
    idx = _strip_argmin(d, e.shape[0])
    q2 = _gather_rows(idx, e)
    i_ref[...] = idx
    out_ref[...] = q1_ref[...] + q2


def kernel(z, embed):
    b, t, dm = z.shape
    k = embed.shape[0]
    n = b * t
    flat = z.reshape(n, dm)
    en_row = jnp.sum(embed * embed, axis=1)[None, :]
    xn1 = jnp.sum(flat * flat, axis=1, keepdims=True)

    grid = (n // _TB,)
    tok_spec = pl.BlockSpec((_TB, dm), lambda i: (i, 0))
    e_spec = pl.BlockSpec((k, dm), lambda i: (0, 0))
    xn_spec = pl.BlockSpec((_TB, 1), lambda i: (i, 0))
    en_spec = pl.BlockSpec((1, k), lambda i: (0, 0))
    idx_spec = pl.BlockSpec((_TB, 1), lambda i: (i, 0))

    i1, q1, r1 = pl.pallas_call(
        _stage1_body,
        grid=grid,
        in_specs=[tok_spec, e_spec, xn_spec, en_spec],
        out_specs=[idx_spec, tok_spec, tok_spec],
        out_shape=[
            jax.ShapeDtypeStruct((n, 1), jnp.int32),
            jax.ShapeDtypeStruct((n, dm), jnp.float32),
            jax.ShapeDtypeStruct((n, dm), jnp.float32),
        ],
    )(flat, embed, xn1, en_row)

    xn2 = jnp.sum(r1 * r1, axis=1, keepdims=True)

    i2, out = pl.pallas_call(
        _stage2_body,
        grid=grid,
        in_specs=[tok_spec, e_spec, xn_spec, en_spec, tok_spec],
        out_specs=[idx_spec, tok_spec],
        out_shape=[
            jax.ShapeDtypeStruct((n, 1), jnp.int32),
            jax.ShapeDtypeStruct((n, dm), jnp.float32),
        ],
    )(r1, embed, xn2, en_row, q1)

    return (out.reshape(b, t, dm), i1.reshape(b, t).astype(jnp.int32),
            i2.reshape(b, t))


# speedup vs baseline: 1.7337x; 1.6344x over previous
import jax
import jax.numpy as jnp
from jax import lax
from jax.experimental import pallas as pl

_STRIPS = 4
_TB = 256
_LANES = 128


def _strip_argmin(d, k):
    tb = d.shape[0]
    w = k // _STRIPS
    lane = lax.broadcasted_iota(jnp.int32, (tb, _LANES), 1)
    acc = jnp.full((tb, 1), jnp.inf, jnp.float32)
    idx = jnp.zeros((tb, 1), jnp.int32)
    for c in range(_STRIPS):
        vacc = d[:, c * w:c * w + _LANES]
        cacc = jnp.zeros((tb, _LANES), jnp.int32)
        for j in range(1, w // _LANES):
            v = d[:, c * w + j * _LANES:c * w + (j + 1) * _LANES]
            win = v < vacc
            vacc = jnp.minimum(vacc, v)
            cacc = jnp.where(win, j, cacc)
        gidx = c * w + cacc * _LANES + lane
        m = jnp.min(vacc, axis=1, keepdims=True)
        ic = jnp.min(jnp.where(vacc == m, gidx, k), axis=1, keepdims=True)
        win = m < acc
        idx = jnp.where(win, ic, idx)
        acc = jnp.where(win, m.astype(jnp.bfloat16).astype(jnp.float32), acc)
    return idx


def _distances(x, e_hi, xn, en_row):
    s = lax.dot_general((-2.0 * x).astype(jnp.bfloat16), e_hi,
                        (((1,), (1,)), ((), ())),
                        preferred_element_type=jnp.float32)
    return (xn + s) + en_row


def _gather_rows(idx, e_hi, e_mid, e_lo):
    iota = lax.broadcasted_iota(jnp.int32, (idx.shape[0], e_hi.shape[0]), 1)
    onehot = (iota == idx).astype(jnp.float32).astype(jnp.bfloat16)
    dims = (((1,), (0,)), ((), ()))
    q_hi = lax.dot_general(onehot, e_hi, dims, preferred_element_type=jnp.float32)
    q_mid = lax.dot_general(onehot, e_mid, dims, preferred_element_type=jnp.float32)
    q_lo = lax.dot_general(onehot, e_lo, dims, preferred_element_type=jnp.float32)
    return (q_hi + q_mid) + q_lo


def _stage1_body(x_ref, ehi_ref, emid_ref, elo_ref, xn_ref, en_ref,
                 i_ref, q_ref, r_ref):
    x = x_ref[...]
    e_hi = ehi_ref[...]
    d = _distances(x, e_hi, xn_ref[...], en_ref[...])
    idx = _strip_argmin(d, e_hi.shape[0])
    q = _gather_rows(idx, e_hi, emid_ref[...], elo_ref[...])
    i_ref[...] = idx
    q_ref[...] = q
    r_ref[...] = x - q


def _stage2_body(r_ref, ehi_ref, emid_ref, elo_ref, xn_ref, en_ref, q1_ref,
                 i_ref, out_ref):
    r = r_ref[...]
    e_hi = ehi_ref[...]
    d = _distances(r, e_hi, xn_ref[...], en_ref[...])
    idx = _strip_argmin(d, e_hi.shape[0])
    q2 = _gather_rows(idx, e_hi, emid_ref[...], elo_ref[...])
    i_ref[...] = idx
    out_ref[...] = q1_ref[...] + q2


def kernel(z, embed):
    b, t, dm = z.shape
    k = embed.shape[0]
    n = b * t
    flat = z.reshape(n, dm)
    en_row = jnp.sum(embed * embed, axis=1)[None, :]
    xn1 = jnp.sum(flat * flat, axis=1, keepdims=True)
    e_hi = embed.astype(jnp.bfloat16)
    e_mid = (embed - e_hi.astype(jnp.float32)).astype(jnp.bfloat16)
    e_lo = (embed - e_hi.astype(jnp.float32)
            - e_mid.astype(jnp.float32)).astype(jnp.bfloat16)

    grid = (n // _TB,)
    tok_spec = pl.BlockSpec((_TB, dm), lambda i: (i, 0))
    e_spec = pl.BlockSpec((k, dm), lambda i: (0, 0))
    xn_spec = pl.BlockSpec((_TB, 1), lambda i: (i, 0))
    en_spec = pl.BlockSpec((1, k), lambda i: (0, 0))
    idx_spec = pl.BlockSpec((_TB, 1), lambda i: (i, 0))

    i1, q1, r1 = pl.pallas_call(
        _stage1_body,
        grid=grid,
        in_specs=[tok_spec, e_spec, e_spec, e_spec, xn_spec, en_spec],
        out_specs=[idx_spec, tok_spec, tok_spec],
        out_shape=[
            jax.ShapeDtypeStruct((n, 1), jnp.int32),
            jax.ShapeDtypeStruct((n, dm), jnp.float32),
            jax.ShapeDtypeStruct((n, dm), jnp.float32),
        ],
    )(flat, e_hi, e_mid, e_lo, xn1, en_row)

    xn2 = jnp.sum(r1 * r1, axis=1, keepdims=True)

    i2, out = pl.pallas_call(
        _stage2_body,
        grid=grid,
        in_specs=[tok_spec, e_spec, e_spec, e_spec, xn_spec, en_spec, tok_spec],
        out_specs=[idx_spec, tok_spec],
        out_shape=[
            jax.ShapeDtypeStruct((n, 1), jnp.int32),
            jax.ShapeDtypeStruct((n, dm), jnp.float32),
        ],
    )(r1, e_hi, e_mid, e_lo, xn2, en_row, q1)

    return out.reshape(b, t, dm), i1.reshape(b, t), i2.reshape(b, t)


# speedup vs baseline: 1.7345x; 1.0004x over previous
import jax
import jax.numpy as jnp
from jax import lax
from jax.experimental import pallas as pl

_STRIPS = 4
_TB = 256
_LANES = 128


def _strip_argmin(d, k):
    tb = d.shape[0]
    w = k // _STRIPS
    lane = lax.broadcasted_iota(jnp.int32, (tb, _LANES), 1)
    acc = jnp.full((tb, 1), jnp.inf, jnp.float32)
    idx = jnp.zeros((tb, 1), jnp.int32)
    for c in range(_STRIPS):
        vacc = d[:, c * w:c * w + _LANES]
        cacc = jnp.zeros((tb, _LANES), jnp.int32)
        for j in range(1, w // _LANES):
            v = d[:, c * w + j * _LANES:c * w + (j + 1) * _LANES]
            win = v < vacc
            vacc = jnp.minimum(vacc, v)
            cacc = jnp.where(win, j, cacc)
        gidx = c * w + cacc * _LANES + lane
        m = jnp.min(vacc, axis=1, keepdims=True)
        ic = jnp.min(jnp.where(vacc == m, gidx, k), axis=1, keepdims=True)
        win = m < acc
        idx = jnp.where(win, ic, idx)
        acc = jnp.where(win, m.astype(jnp.bfloat16).astype(jnp.float32), acc)
    return idx


def _distances(x, e_hi, xn, en_row):
    s = lax.dot_general((-2.0 * x).astype(jnp.bfloat16), e_hi,
                        (((1,), (1,)), ((), ())),
                        preferred_element_type=jnp.float32)
    return (xn + s) + en_row


def _gather_rows(idx, e_hi, e_mid, e_lo):
    iota = lax.broadcasted_iota(jnp.int32, (idx.shape[0], e_hi.shape[0]), 1)
    onehot = (iota == idx).astype(jnp.float32).astype(jnp.bfloat16)
    dims = (((1,), (0,)), ((), ()))
    q_hi = lax.dot_general(onehot, e_hi, dims, preferred_element_type=jnp.float32)
    q_mid = lax.dot_general(onehot, e_mid, dims, preferred_element_type=jnp.float32)
    q_lo = lax.dot_general(onehot, e_lo, dims, preferred_element_type=jnp.float32)
    return (q_hi + q_mid) + q_lo


def _split_body(e_ref, hi_ref, mid_ref, lo_ref):
    e = e_ref[...]
    hi = e.astype(jnp.bfloat16)
    rem = e - hi.astype(jnp.float32)
    mid = rem.astype(jnp.bfloat16)
    lo = (rem - mid.astype(jnp.float32)).astype(jnp.bfloat16)
    hi_ref[...] = hi
    mid_ref[...] = mid
    lo_ref[...] = lo


def _stage1_body(x_ref, ehi_ref, emid_ref, elo_ref, xn_ref, en_ref,
                 i_ref, q_ref, r_ref):
    x = x_ref[...]
    e_hi = ehi_ref[...]
    d = _distances(x, e_hi, xn_ref[...], en_ref[...])
    idx = _strip_argmin(d, e_hi.shape[0])
    q = _gather_rows(idx, e_hi, emid_ref[...], elo_ref[...])
    i_ref[...] = idx
    q_ref[...] = q
    r_ref[...] = x - q


def _stage2_body(r_ref, ehi_ref, emid_ref, elo_ref, xn_ref, en_ref, q1_ref,
                 i_ref, out_ref):
    r = r_ref[...]
    e_hi = ehi_ref[...]
    d = _distances(r, e_hi, xn_ref[...], en_ref[...])
    idx = _strip_argmin(d, e_hi.shape[0])
    q2 = _gather_rows(idx, e_hi, emid_ref[...], elo_ref[...])
    i_ref[...] = idx
    out_ref[...] = q1_ref[...] + q2


def kernel(z, embed):
    b, t, dm = z.shape
    k = embed.shape[0]
    n = b * t
    flat = z.reshape(n, dm)
    en_row = jnp.sum(embed * embed, axis=1)[None, :]
    xn1 = jnp.sum(flat * flat, axis=1, keepdims=True)
    e_hi, e_mid, e_lo = pl.pallas_call(
        _split_body,
        out_shape=[jax.ShapeDtypeStruct((k, dm), jnp.bfloat16)] * 3,
    )(embed)

    grid = (n // _TB,)
    tok_spec = pl.BlockSpec((_TB, dm), lambda i: (i, 0))
    e_spec = pl.BlockSpec((k, dm), lambda i: (0, 0))
    xn_spec = pl.BlockSpec((_TB, 1), lambda i: (i, 0))
    en_spec = pl.BlockSpec((1, k), lambda i: (0, 0))
    idx_spec = pl.BlockSpec((_TB, 1), lambda i: (i, 0))

    i1, q1, r1 = pl.pallas_call(
        _stage1_body,
        grid=grid,
        in_specs=[tok_spec, e_spec, e_spec, e_spec, xn_spec, en_spec],
        out_specs=[idx_spec, tok_spec, tok_spec],
        out_shape=[
            jax.ShapeDtypeStruct((n, 1), jnp.int32),
            jax.ShapeDtypeStruct((n, dm), jnp.float32),
            jax.ShapeDtypeStruct((n, dm), jnp.float32),
        ],
    )(flat, e_hi, e_mid, e_lo, xn1, en_row)

    xn2 = jnp.sum(r1 * r1, axis=1, keepdims=True)

    i2, out = pl.pallas_call(
        _stage2_body,
        grid=grid,
        in_specs=[tok_spec, e_spec, e_spec, e_spec, xn_spec, en_spec, tok_spec],
        out_specs=[idx_spec, tok_spec],
        out_shape=[
            jax.ShapeDtypeStruct((n, 1), jnp.int32),
            jax.ShapeDtypeStruct((n, dm), jnp.float32),
        ],
    )(r1, e_hi, e_mid, e_lo, xn2, en_row, q1)

    return out.reshape(b, t, dm), i1.reshape(b, t), i2.reshape(b, t)


# speedup vs baseline: 2.6265x; 1.5143x over previous
import jax
import jax.numpy as jnp
from jax import lax
from jax.experimental import pallas as pl

_STRIPS = 4
_TB = 256
_LANES = 128


def _strip_argmin(d, k):
    tb = d.shape[0]
    w = k // _STRIPS
    lane = lax.broadcasted_iota(jnp.int32, (tb, _LANES), 1)
    acc = jnp.full((tb, 1), jnp.inf, jnp.float32)
    idx = jnp.zeros((tb, 1), jnp.int32)
    for c in range(_STRIPS):
        vacc = d[:, c * w:c * w + _LANES]
        cacc = jnp.zeros((tb, _LANES), jnp.int32)
        for j in range(1, w // _LANES):
            v = d[:, c * w + j * _LANES:c * w + (j + 1) * _LANES]
            win = v < vacc
            vacc = jnp.minimum(vacc, v)
            cacc = jnp.where(win, j, cacc)
        gidx = c * w + cacc * _LANES + lane
        m = jnp.min(vacc, axis=1, keepdims=True)
        ic = jnp.min(jnp.where(vacc == m, gidx, k), axis=1, keepdims=True)
        win = m < acc
        idx = jnp.where(win, ic, idx)
        acc = jnp.where(win, m.astype(jnp.bfloat16).astype(jnp.float32), acc)
    return idx


def _distances(x, e_hi, xn, en_row):
    s = lax.dot_general((-2.0 * x).astype(jnp.bfloat16), e_hi,
                        (((1,), (1,)), ((), ())),
                        preferred_element_type=jnp.float32)
    return (xn + s) + en_row


def _gather_rows(idx, e_cat, dm):
    iota = lax.broadcasted_iota(jnp.int32, (idx.shape[0], e_cat.shape[0]), 1)
    onehot = (iota == idx).astype(jnp.float32).astype(jnp.bfloat16)
    q_cat = lax.dot_general(onehot, e_cat, (((1,), (0,)), ((), ())),
                            preferred_element_type=jnp.float32)
    return (q_cat[:, :dm] + q_cat[:, dm:2 * dm]) + q_cat[:, 2 * dm:]


def _split_body(e_ref, hi_ref, cat_ref):
    e = e_ref[...]
    dm = e.shape[1]
    hi = e.astype(jnp.bfloat16)
    rem = e - hi.astype(jnp.float32)
    mid = rem.astype(jnp.bfloat16)
    lo = (rem - mid.astype(jnp.float32)).astype(jnp.bfloat16)
    hi_ref[...] = hi
    cat_ref[:, :dm] = hi
    cat_ref[:, dm:2 * dm] = mid
    cat_ref[:, 2 * dm:] = lo


def _stage1_body(x_ref, ehi_ref, ecat_ref, xn_ref, en_ref,
                 i_ref, q_ref, r_ref):
    x = x_ref[...]
    e_hi = ehi_ref[...]
    d = _distances(x, e_hi, xn_ref[...], en_ref[...])
    idx = _strip_argmin(d, e_hi.shape[0])
    q = _gather_rows(idx, ecat_ref[...], x.shape[1])
    i_ref[...] = idx
    q_ref[...] = q
    r_ref[...] = x - q


def _stage2_body(r_ref, ehi_ref, ecat_ref, xn_ref, en_ref, q1_ref,
                 i_ref, out_ref):
    r = r_ref[...]
    e_hi = ehi_ref[...]
    d = _distances(r, e_hi, xn_ref[...], en_ref[...])
    idx = _strip_argmin(d, e_hi.shape[0])
    q2 = _gather_rows(idx, ecat_ref[...], r.shape[1])
    i_ref[...] = idx
    out_ref[...] = q1_ref[...] + q2


def kernel(z, embed):
    b, t, dm = z.shape
    k = embed.shape[0]
    n = b * t
    flat = z.reshape(n, dm)
    en_row = jnp.sum(embed * embed, axis=1)[None, :]
    xn1 = jnp.sum(flat * flat, axis=1, keepdims=True)
    e_hi, e_cat = pl.pallas_call(
        _split_body,
        out_shape=[jax.ShapeDtypeStruct((k, dm), jnp.bfloat16),
                   jax.ShapeDtypeStruct((k, 3 * dm), jnp.bfloat16)],
    )(embed)

    grid = (n // _TB,)
    tok_spec = pl.BlockSpec((_TB, dm), lambda i: (i, 0))
    e_spec = pl.BlockSpec((k, dm), lambda i: (0, 0))
    ecat_spec = pl.BlockSpec((k, 3 * dm), lambda i: (0, 0))
    xn_spec = pl.BlockSpec((_TB, 1), lambda i: (i, 0))
    en_spec = pl.BlockSpec((1, k), lambda i: (0, 0))
    idx_spec = pl.BlockSpec((_TB, 1), lambda i: (i, 0))

    i1, q1, r1 = pl.pallas_call(
        _stage1_body,
        grid=grid,
        in_specs=[tok_spec, e_spec, ecat_spec, xn_spec, en_spec],
        out_specs=[idx_spec, tok_spec, tok_spec],
        out_shape=[
            jax.ShapeDtypeStruct((n, 1), jnp.int32),
            jax.ShapeDtypeStruct((n, dm), jnp.float32),
            jax.ShapeDtypeStruct((n, dm), jnp.float32),
        ],
    )(flat, e_hi, e_cat, xn1, en_row)

    xn2 = jnp.sum(r1 * r1, axis=1, keepdims=True)

    i2, out = pl.pallas_call(
        _stage2_body,
        grid=grid,
        in_specs=[tok_spec, e_spec, ecat_spec, xn_spec, en_spec, tok_spec],
        out_specs=[idx_spec, tok_spec],
        out_shape=[
            jax.ShapeDtypeStruct((n, 1), jnp.int32),
            jax.ShapeDtypeStruct((n, dm), jnp.float32),
        ],
    )(r1, e_hi, e_cat, xn2, en_row, q1)

    return out.reshape(b, t, dm), i1.reshape(b, t), i2.reshape(b, t)
